# baseline (speedup 1.0000x reference)
import functools

import jax
import jax.numpy as jnp
from jax import lax
from jax.experimental import pallas as pl
from jax.experimental.pallas import tpu as pltpu
from jax.experimental.pallas import tpu_sc as plsc

N = 10000
D = 128
E = 320000
NC = 2
NS = 16
NW = NC * NS
EPW = E // NW
K = 128
CHF = EPW // K
TAIL = EPW - CHF * K
NB = 2
NI = 6
ACC_R = 10240
RPT = ACC_R // NS
G_TAIL = N - (NS - 1) * RPT

_mesh = plsc.VectorSubcoreMesh(core_axis_name="c", subcore_axis_name="s")


@functools.partial(
    pl.kernel,
    mesh=_mesh,
    out_type=jax.ShapeDtypeStruct((NC, ACC_R), jnp.float32),
    scratch_types=[
        pltpu.VMEM((3, K), jnp.int32),
        pltpu.VMEM((16,), jnp.int32),
        pltpu.VMEM((K,), jnp.float32),
        pltpu.VMEM((640,), jnp.float32),
        pltpu.VMEM_SHARED((ACC_R,), jnp.float32),
    ] + [pltpu.SemaphoreType.DMA] * 3,
)
def _sc_degree(ei_hbm, out_hbm, idx_v, tidx_v, ones_v, init_v, dacc,
               si0, si1, si2):
    si = (si0, si1, si2)
    cid = lax.axis_index("c")
    sid = lax.axis_index("s")
    wid = sid * NC + cid
    for b in range(3):
        pltpu.async_copy(ei_hbm.at[1, wid, pl.ds(b * K, K)],
                         idx_v.at[b], si[b])
    one = jnp.float32(1.0)
    for i in range(K // 16):
        ones_v[pl.ds(16 * i, 16)] = jnp.full((16,), one, jnp.float32)
    init = jnp.where(cid == 0, one, jnp.float32(0.0))
    for i in range(640 // 16):
        init_v[pl.ds(16 * i, 16)] = jnp.full((16,), init, jnp.float32)
    pltpu.sync_copy(init_v.at[pl.ds(0, RPT)], dacc.at[pl.ds(sid * RPT, RPT)])
    plsc.subcore_barrier()

    def body(t, carry):
        j0 = 3 * t
        for u in range(3):
            j = j0 + u
            pltpu.make_async_copy(ei_hbm.at[1, wid, pl.ds(0, K)],
                                  idx_v.at[u], si[u]).wait()
            pltpu.sync_copy(ones_v, dacc.at[idx_v.at[u]], add=True)

            @pl.when(j + 3 < CHF)
            def _():
                pltpu.async_copy(ei_hbm.at[1, wid, pl.ds((j + 3) * K, K)],
                                 idx_v.at[u], si[u])
        return carry

    lax.fori_loop(0, CHF // 3, body, 0)
    pltpu.sync_copy(ei_hbm.at[1, wid, pl.ds(CHF * K, TAIL)], tidx_v)
    pltpu.sync_copy(ones_v.at[pl.ds(0, TAIL)], dacc.at[tidx_v], add=True)
    plsc.subcore_barrier()
    pltpu.sync_copy(dacc.at[pl.ds(sid * RPT, RPT)],
                    out_hbm.at[cid, pl.ds(sid * RPT, RPT)])


@functools.partial(
    pl.kernel,
    mesh=_mesh,
    out_type=jax.ShapeDtypeStruct((NC, ACC_R, D), jnp.float32),
    scratch_types=[
        pltpu.VMEM((NI, K), jnp.int32),
        pltpu.VMEM((NI, K), jnp.int32),
        pltpu.VMEM((16,), jnp.int32),
        pltpu.VMEM((16,), jnp.int32),
        pltpu.VMEM((NB, K, D), jnp.float32),
        pltpu.VMEM_SHARED((ACC_R, D), jnp.float32),
    ] + [pltpu.SemaphoreType.DMA] * (2 * NI + 2 * NB),
)
def _sc_scatter(g_hbm, ei_hbm, out_hbm,
                sidx_v, didx_v, tsrc_v, tdst_v, rows_v, acc,
                sa0, sa1, sa2, sa3, sa4, sa5,
                sb0, sb1, sb2, sb3, sb4, sb5,
                sg0, sg1, ss0, ss1):
    sa = (sa0, sa1, sa2, sa3, sa4, sa5)
    sb = (sb0, sb1, sb2, sb3, sb4, sb5)
    sg = (sg0, sg1)
    ss = (ss0, ss1)
    cid = lax.axis_index("c")
    sid = lax.axis_index("s")
    wid = sid * NC + cid

    for b in range(NI):
        pltpu.async_copy(ei_hbm.at[0, wid, pl.ds(b * K, K)],
                         sidx_v.at[b], sa[b])
        pltpu.async_copy(ei_hbm.at[1, wid, pl.ds(b * K, K)],
                         didx_v.at[b], sb[b])

    @pl.when(cid == 0)
    def _():
        @pl.when(sid < NS - 1)
        def _():
            pltpu.sync_copy(g_hbm.at[pl.ds(sid * RPT, RPT)],
                            acc.at[pl.ds(sid * RPT, RPT)])

        @pl.when(sid == NS - 1)
        def _():
            pltpu.sync_copy(g_hbm.at[pl.ds((NS - 1) * RPT, G_TAIL)],
                            acc.at[pl.ds((NS - 1) * RPT, G_TAIL)])
            pltpu.sync_copy(g_hbm.at[pl.ds(0, ACC_R - N)],
                            acc.at[pl.ds(N, ACC_R - N)])

    @pl.when(cid == 1)
    def _():
        def zrow(i, carry):
            for c in range(D // 16):
                rows_v[0, i, pl.ds(16 * c, 16)] = jnp.zeros((16,), jnp.float32)
            return carry

        lax.fori_loop(0, K, zrow, 0)
        for r in range(RPT // K):
            pltpu.sync_copy(rows_v.at[0],
                            acc.at[pl.ds(sid * RPT + r * K, K)])

    plsc.subcore_barrier()

    def body(t, carry):
        j0 = NI * t
        for u in range(NI):
            j = j0 + u
            rb = u % NB
            pb = (u - 1) % NI
            qb = (u - 2) % NI
            pltpu.make_async_copy(ei_hbm.at[0, wid, pl.ds(0, K)],
                                  sidx_v.at[u], sa[u]).wait()
            pltpu.make_async_copy(ei_hbm.at[1, wid, pl.ds(0, K)],
                                  didx_v.at[u], sb[u]).wait()

            def wait_prev_scatter():
                pltpu.make_async_copy(rows_v.at[rb],
                                      acc.at[didx_v.at[qb]], ss[rb]).wait()

            if u < 2:
                pl.when(t > 0)(wait_prev_scatter)
            else:
                wait_prev_scatter()
            pltpu.async_copy(g_hbm.at[sidx_v.at[u]], rows_v.at[rb], sg[rb])

            def service_prev():
                pltpu.make_async_copy(g_hbm.at[sidx_v.at[pb]],
                                      rows_v.at[1 - rb], sg[1 - rb]).wait()
                pltpu.async_copy(rows_v.at[1 - rb],
                                 acc.at[didx_v.at[pb]], ss[1 - rb], add=True)

            if u == 0:
                pl.when(t > 0)(service_prev)
            else:
                service_prev()

            @pl.when(jnp.logical_and(j >= 2, j + NI - 2 < CHF))
            def _():
                pltpu.async_copy(ei_hbm.at[0, wid, pl.ds((j + NI - 2) * K, K)],
                                 sidx_v.at[qb], sa[qb])
                pltpu.async_copy(ei_hbm.at[1, wid, pl.ds((j + NI - 2) * K, K)],
                                 didx_v.at[qb], sb[qb])
        return carry

    lax.fori_loop(0, CHF // NI, body, 0)
    lb = (CHF - 1) % NB
    pltpu.make_async_copy(g_hbm.at[sidx_v.at[NI - 1]],
                          rows_v.at[lb], sg[lb]).wait()
    pltpu.async_copy(rows_v.at[lb], acc.at[didx_v.at[NI - 1]], ss[lb],
                     add=True)
    pltpu.make_async_copy(rows_v.at[1 - lb],
                          acc.at[didx_v.at[NI - 2]], ss[1 - lb]).wait()
    pltpu.make_async_copy(rows_v.at[lb],
                          acc.at[didx_v.at[NI - 1]], ss[lb]).wait()
    pltpu.sync_copy(ei_hbm.at[0, wid, pl.ds(CHF * K, TAIL)], tsrc_v)
    pltpu.sync_copy(ei_hbm.at[1, wid, pl.ds(CHF * K, TAIL)], tdst_v)
    pltpu.async_copy(g_hbm.at[tsrc_v], rows_v.at[0, pl.ds(0, TAIL)],
                     sg[0]).wait()
    pltpu.sync_copy(rows_v.at[0, pl.ds(0, TAIL)], acc.at[tdst_v], add=True)
    plsc.subcore_barrier()
    pltpu.sync_copy(acc.at[pl.ds(sid * RPT, RPT)],
                    out_hbm.at[cid, pl.ds(sid * RPT, RPT)])


RB = 2000
_GRID = N // RB


def _row_spec(shape_tail):
    return pl.BlockSpec((RB,) + shape_tail, lambda i: (i,) + (0,) * len(shape_tail))


def _full_spec(shape):
    return pl.BlockSpec(shape, lambda i: (0,) * len(shape))


def _s_spec():
    return pl.BlockSpec((2, RB, D), lambda i: (0, i, 0))


def _mm_body(x_ref, w_ref, o_ref):
    o_ref[...] = jnp.dot(x_ref[...], w_ref[...],
                         preferred_element_type=jnp.float32)


def _tc_matmul(x, w):
    return pl.pallas_call(
        _mm_body,
        grid=(_GRID,),
        in_specs=[_row_spec((D,)), _full_spec((D, D))],
        out_specs=_row_spec((D,)),
        out_shape=jax.ShapeDtypeStruct((N, D), jnp.float32),
    )(x, w)


def _dinv_body(degp_ref, dinv_ref):
    deg = degp_ref[0:1, :N] + degp_ref[1:2, :N]
    dinv_ref[...] = jnp.transpose(lax.rsqrt(deg), (1, 0))


def _tc_dinv(degp):
    return pl.pallas_call(
        _dinv_body,
        out_shape=jax.ShapeDtypeStruct((N, 1), jnp.float32),
    )(degp)


def _scale_body(p_ref, dinv_ref, o_ref):
    o_ref[...] = p_ref[...] * dinv_ref[...]


def _tc_scale(p, dinv):
    return pl.pallas_call(
        _scale_body,
        grid=(_GRID,),
        in_specs=[_row_spec((D,)), _row_spec((1,))],
        out_specs=_row_spec((D,)),
        out_shape=jax.ShapeDtypeStruct((N, D), jnp.float32),
    )(p, dinv)


def _mid_body(s_ref, dinv_ref, b_ref, w_ref, o_ref):
    s = s_ref[0] + s_ref[1]
    h = jnp.maximum(dinv_ref[...] * s + b_ref[...], 0.0)
    o_ref[...] = dinv_ref[...] * jnp.dot(h, w_ref[...],
                                         preferred_element_type=jnp.float32)


def _tc_mid(S, dinv, b, w):
    return pl.pallas_call(
        _mid_body,
        grid=(_GRID,),
        in_specs=[_s_spec(), _row_spec((1,)), _full_spec((1, D)),
                  _full_spec((D, D))],
        out_specs=_row_spec((D,)),
        out_shape=jax.ShapeDtypeStruct((N, D), jnp.float32),
    )(S, dinv, b, w)


def _out_body(s_ref, dinv_ref, b_ref, wfc_ref, bfc_ref, o_ref):
    s = s_ref[0] + s_ref[1]
    h = jnp.maximum(dinv_ref[...] * s + b_ref[...], 0.0)
    logit = jnp.dot(h, wfc_ref[...],
                    preferred_element_type=jnp.float32) + bfc_ref[...]
    m = jnp.max(logit, axis=1, keepdims=True)
    ssum = jnp.sum(jnp.exp(logit - m), axis=1, keepdims=True)
    o_ref[...] = logit - m - jnp.log(ssum)


def _tc_out(S, dinv, b, wfc, bfc):
    return pl.pallas_call(
        _out_body,
        grid=(_GRID,),
        in_specs=[_s_spec(), _row_spec((1,)), _full_spec((1, D)),
                  _full_spec((D, 2)), _full_spec((1, 2))],
        out_specs=_row_spec((2,)),
        out_shape=jax.ShapeDtypeStruct((N, 2), jnp.float32),
    )(S, dinv, b, wfc, bfc)


def kernel(x, edge_index, W1, b1, W2, b2, Wfc, bfc):
    ei = edge_index.reshape(2, NW, EPW)

    degp = _sc_degree(ei)
    p1 = _tc_matmul(x, W1)
    dinv = _tc_dinv(degp)
    g1 = _tc_scale(p1, dinv)
    S1 = _sc_scatter(g1, ei)
    g2 = _tc_mid(S1, dinv, b1.reshape(1, D), W2)
    S2 = _sc_scatter(g2, ei)
    return _tc_out(S2, dinv, b2.reshape(1, D), Wfc, bfc.reshape(1, 2))


# speedup vs baseline: 1.0173x; 1.0173x over previous
import functools

import jax
import jax.numpy as jnp
from jax import lax
from jax.experimental import pallas as pl
from jax.experimental.pallas import tpu as pltpu
from jax.experimental.pallas import tpu_sc as plsc

N = 10000
D = 128
E = 320000
NC = 2
NS = 16
NW = NC * NS
EPW = E // NW
K = 128
CHF = EPW // K
TAIL = EPW - CHF * K
NB = 2
NI = 6
ACC_R = 10240
RPT = ACC_R // NS
G_TAIL = N - (NS - 1) * RPT

_mesh = plsc.VectorSubcoreMesh(core_axis_name="c", subcore_axis_name="s")


@functools.partial(
    pl.kernel,
    mesh=_mesh,
    out_type=jax.ShapeDtypeStruct((NC, ACC_R), jnp.float32),
    scratch_types=[
        pltpu.VMEM((3, K), jnp.int32),
        pltpu.VMEM((16,), jnp.int32),
        pltpu.VMEM((K,), jnp.float32),
        pltpu.VMEM((640,), jnp.float32),
        pltpu.VMEM_SHARED((ACC_R,), jnp.float32),
    ] + [pltpu.SemaphoreType.DMA] * 3,
)
def _sc_degree(ei_hbm, out_hbm, idx_v, tidx_v, ones_v, init_v, dacc,
               si0, si1, si2):
    si = (si0, si1, si2)
    cid = lax.axis_index("c")
    sid = lax.axis_index("s")
    wid = sid * NC + cid
    for b in range(3):
        pltpu.async_copy(ei_hbm.at[1, wid, pl.ds(b * K, K)],
                         idx_v.at[b], si[b])
    one = jnp.float32(1.0)
    for i in range(K // 16):
        ones_v[pl.ds(16 * i, 16)] = jnp.full((16,), one, jnp.float32)
    init = jnp.where(cid == 0, one, jnp.float32(0.0))
    for i in range(640 // 16):
        init_v[pl.ds(16 * i, 16)] = jnp.full((16,), init, jnp.float32)
    pltpu.sync_copy(init_v.at[pl.ds(0, RPT)], dacc.at[pl.ds(sid * RPT, RPT)])
    plsc.subcore_barrier()

    def body(t, carry):
        j0 = 3 * t
        for u in range(3):
            j = j0 + u
            pltpu.make_async_copy(ei_hbm.at[1, wid, pl.ds(0, K)],
                                  idx_v.at[u], si[u]).wait()
            pltpu.sync_copy(ones_v, dacc.at[idx_v.at[u]], add=True)

            @pl.when(j + 3 < CHF)
            def _():
                pltpu.async_copy(ei_hbm.at[1, wid, pl.ds((j + 3) * K, K)],
                                 idx_v.at[u], si[u])
        return carry

    lax.fori_loop(0, CHF // 3, body, 0)
    pltpu.sync_copy(ei_hbm.at[1, wid, pl.ds(CHF * K, TAIL)], tidx_v)
    pltpu.sync_copy(ones_v.at[pl.ds(0, TAIL)], dacc.at[tidx_v], add=True)
    plsc.subcore_barrier()
    pltpu.sync_copy(dacc.at[pl.ds(sid * RPT, RPT)],
                    out_hbm.at[cid, pl.ds(sid * RPT, RPT)])


@functools.partial(
    pl.kernel,
    mesh=_mesh,
    out_type=jax.ShapeDtypeStruct((NC, ACC_R, D), jnp.float32),
    scratch_types=[
        pltpu.VMEM((NI, K), jnp.int32),
        pltpu.VMEM((NI, K), jnp.int32),
        pltpu.VMEM((16,), jnp.int32),
        pltpu.VMEM((16,), jnp.int32),
        pltpu.VMEM((NB, K, D), jnp.float32),
        pltpu.VMEM_SHARED((ACC_R, D), jnp.float32),
    ] + [pltpu.SemaphoreType.DMA] * (2 * NI + 2 * NB),
)
def _sc_scatter(g_hbm, ei_hbm, out_hbm,
                sidx_v, didx_v, tsrc_v, tdst_v, rows_v, acc,
                sa0, sa1, sa2, sa3, sa4, sa5,
                sb0, sb1, sb2, sb3, sb4, sb5,
                sg0, sg1, ss0, ss1):
    sa = (sa0, sa1, sa2, sa3, sa4, sa5)
    sb = (sb0, sb1, sb2, sb3, sb4, sb5)
    sg = (sg0, sg1)
    ss = (ss0, ss1)
    cid = lax.axis_index("c")
    sid = lax.axis_index("s")
    wid = sid * NC + cid

    for b in range(NI):
        pltpu.async_copy(ei_hbm.at[0, wid, pl.ds(b * K, K)],
                         sidx_v.at[b], sa[b])
        pltpu.async_copy(ei_hbm.at[1, wid, pl.ds(b * K, K)],
                         didx_v.at[b], sb[b])

    @pl.when(cid == 0)
    def _():
        @pl.when(sid < NS - 1)
        def _():
            pltpu.sync_copy(g_hbm.at[pl.ds(sid * RPT, RPT)],
                            acc.at[pl.ds(sid * RPT, RPT)])

        @pl.when(sid == NS - 1)
        def _():
            pltpu.sync_copy(g_hbm.at[pl.ds((NS - 1) * RPT, G_TAIL)],
                            acc.at[pl.ds((NS - 1) * RPT, G_TAIL)])
            pltpu.sync_copy(g_hbm.at[pl.ds(0, ACC_R - N)],
                            acc.at[pl.ds(N, ACC_R - N)])

    @pl.when(cid == 1)
    def _():
        def zrow(i, carry):
            for c in range(D // 16):
                rows_v[0, i, pl.ds(16 * c, 16)] = jnp.zeros((16,), jnp.float32)
            return carry

        lax.fori_loop(0, K, zrow, 0)
        for r in range(RPT // K):
            pltpu.sync_copy(rows_v.at[0],
                            acc.at[pl.ds(sid * RPT + r * K, K)])

    plsc.subcore_barrier()

    def body(t, carry):
        j0 = NI * t
        for u in range(NI):
            j = j0 + u
            rb = u % NB
            pb = (u - 1) % NI
            qb = (u - 2) % NI
            pltpu.make_async_copy(ei_hbm.at[0, wid, pl.ds(0, K)],
                                  sidx_v.at[u], sa[u]).wait()
            pltpu.make_async_copy(ei_hbm.at[1, wid, pl.ds(0, K)],
                                  didx_v.at[u], sb[u]).wait()

            def wait_prev_scatter():
                pltpu.make_async_copy(rows_v.at[rb],
                                      acc.at[didx_v.at[qb]], ss[rb]).wait()

            if u < 2:
                pl.when(t > 0)(wait_prev_scatter)
            else:
                wait_prev_scatter()
            pltpu.async_copy(g_hbm.at[sidx_v.at[u]], rows_v.at[rb], sg[rb])

            def service_prev():
                pltpu.make_async_copy(g_hbm.at[sidx_v.at[pb]],
                                      rows_v.at[1 - rb], sg[1 - rb]).wait()
                pltpu.async_copy(rows_v.at[1 - rb],
                                 acc.at[didx_v.at[pb]], ss[1 - rb], add=True)

            if u == 0:
                pl.when(t > 0)(service_prev)
            else:
                service_prev()

            @pl.when(jnp.logical_and(j >= 2, j + NI - 2 < CHF))
            def _():
                pltpu.async_copy(ei_hbm.at[0, wid, pl.ds((j + NI - 2) * K, K)],
                                 sidx_v.at[qb], sa[qb])
                pltpu.async_copy(ei_hbm.at[1, wid, pl.ds((j + NI - 2) * K, K)],
                                 didx_v.at[qb], sb[qb])
        return carry

    lax.fori_loop(0, CHF // NI, body, 0)
    lb = (CHF - 1) % NB
    pltpu.make_async_copy(g_hbm.at[sidx_v.at[NI - 1]],
                          rows_v.at[lb], sg[lb]).wait()
    pltpu.async_copy(rows_v.at[lb], acc.at[didx_v.at[NI - 1]], ss[lb],
                     add=True)
    pltpu.make_async_copy(rows_v.at[1 - lb],
                          acc.at[didx_v.at[NI - 2]], ss[1 - lb]).wait()
    pltpu.make_async_copy(rows_v.at[lb],
                          acc.at[didx_v.at[NI - 1]], ss[lb]).wait()
    pltpu.sync_copy(ei_hbm.at[0, wid, pl.ds(CHF * K, TAIL)], tsrc_v)
    pltpu.sync_copy(ei_hbm.at[1, wid, pl.ds(CHF * K, TAIL)], tdst_v)
    pltpu.async_copy(g_hbm.at[tsrc_v], rows_v.at[0, pl.ds(0, TAIL)],
                     sg[0]).wait()
    pltpu.sync_copy(rows_v.at[0, pl.ds(0, TAIL)], acc.at[tdst_v], add=True)
    plsc.subcore_barrier()
    pltpu.sync_copy(acc.at[pl.ds(sid * RPT, RPT)],
                    out_hbm.at[cid, pl.ds(sid * RPT, RPT)])


RB = 2000
_GRID = N // RB


def _row_spec(shape_tail):
    return pl.BlockSpec((RB,) + shape_tail, lambda i: (i,) + (0,) * len(shape_tail))


def _full_spec(shape):
    return pl.BlockSpec(shape, lambda i: (0,) * len(shape))


def _s_spec():
    return pl.BlockSpec((2, RB, D), lambda i: (0, i, 0))


def _mm_body(x_ref, w_ref, o_ref):
    o_ref[...] = jnp.dot(x_ref[...], w_ref[...],
                         preferred_element_type=jnp.float32)


def _tc_matmul(x, w):
    return pl.pallas_call(
        _mm_body,
        grid=(_GRID,),
        in_specs=[_row_spec((D,)), _full_spec((D, D))],
        out_specs=_row_spec((D,)),
        out_shape=jax.ShapeDtypeStruct((N, D), jnp.float32),
    )(x, w)


def _scale_body(p_ref, degp_ref, o_ref, dinv_ref):
    deg = degp_ref[0:1, :N] + degp_ref[1:2, :N]
    dinv = jnp.transpose(lax.rsqrt(deg), (1, 0))
    dinv_ref[...] = dinv
    o_ref[...] = p_ref[...] * dinv


def _tc_scale(p, degp):
    return pl.pallas_call(
        _scale_body,
        out_shape=(jax.ShapeDtypeStruct(p.shape, jnp.float32),
                   jax.ShapeDtypeStruct((N, 1), jnp.float32)),
    )(p, degp)


def _mid_body(s_ref, dinv_ref, b_ref, w_ref, o_ref):
    s = s_ref[0] + s_ref[1]
    h = jnp.maximum(dinv_ref[...] * s + b_ref[...], 0.0)
    o_ref[...] = dinv_ref[...] * jnp.dot(h, w_ref[...],
                                         preferred_element_type=jnp.float32)


def _tc_mid(S, dinv, b, w):
    return pl.pallas_call(
        _mid_body,
        grid=(_GRID,),
        in_specs=[_s_spec(), _row_spec((1,)), _full_spec((1, D)),
                  _full_spec((D, D))],
        out_specs=_row_spec((D,)),
        out_shape=jax.ShapeDtypeStruct((N, D), jnp.float32),
    )(S, dinv, b, w)


def _out_body(s_ref, dinv_ref, b_ref, wfc_ref, bfc_ref, o_ref):
    s = s_ref[0] + s_ref[1]
    h = jnp.maximum(dinv_ref[...] * s + b_ref[...], 0.0)
    logit = jnp.dot(h, wfc_ref[...],
                    preferred_element_type=jnp.float32) + bfc_ref[...]
    m = jnp.max(logit, axis=1, keepdims=True)
    ssum = jnp.sum(jnp.exp(logit - m), axis=1, keepdims=True)
    o_ref[...] = logit - m - jnp.log(ssum)


def _tc_out(S, dinv, b, wfc, bfc):
    return pl.pallas_call(
        _out_body,
        grid=(_GRID,),
        in_specs=[_s_spec(), _row_spec((1,)), _full_spec((1, D)),
                  _full_spec((D, 2)), _full_spec((1, 2))],
        out_specs=_row_spec((2,)),
        out_shape=jax.ShapeDtypeStruct((N, 2), jnp.float32),
    )(S, dinv, b, wfc, bfc)


def kernel(x, edge_index, W1, b1, W2, b2, Wfc, bfc):
    ei = edge_index.reshape(2, NW, EPW)

    degp = _sc_degree(ei)
    p1 = _tc_matmul(x, W1)
    g1, dinv = _tc_scale(p1, degp)
    S1 = _sc_scatter(g1, ei)
    g2 = _tc_mid(S1, dinv, b1.reshape(1, D), W2)
    S2 = _sc_scatter(g2, ei)
    return _tc_out(S2, dinv, b2.reshape(1, D), Wfc, bfc.reshape(1, 2))
